# baseline (speedup 1.0000x reference)
import functools

import jax
import jax.numpy as jnp
from jax import lax
from jax.experimental import pallas as pl
from jax.experimental.pallas import tpu as pltpu
from jax.experimental.pallas import tpu_sc as plsc

NC = 2
NS = 16
NW = NC * NS
L = 16

VOCAB = 1000
VPAD = 1024
NBANDS = VOCAB // 8
ICH = 128
NSTG = 2


def _lse_body(tab_ref, lse_ref):
    x = tab_ref[...]
    m = jnp.max(x, axis=1, keepdims=True)
    s = jnp.sum(jnp.exp(x - m), axis=1, keepdims=True)
    lse_ref[...] = m + jnp.log(s)


def _row_lse(table):
    return pl.pallas_call(
        _lse_body,
        out_shape=jax.ShapeDtypeStruct((VOCAB, 1), jnp.float32),
    )(table)


def _gather_t_body(idx_hbm, tabt_hbm, outt_hbm, idx_v, tab_v, stg_v, sems):
    n = idx_v.shape[0]
    nch = n // ICH
    wid = lax.axis_index("s") * NC + lax.axis_index("c")

    pltpu.sync_copy(idx_hbm, idx_v)

    def out_rect(b, ch):
        return outt_hbm.at[pl.ds(b * 8, 8), pl.ds(ch * ICH, ICH)]

    def compute_tile(s, ch):
        for g in range(ICH // L):
            i16 = idx_v[pl.ds(ch * ICH + g * L, L)]
            for c in range(8):
                v16 = plsc.load_gather(tab_v, [i16 + c * VPAD])
                stg_v[s, c, pl.ds(g * L, L)] = v16

    def band_loop(b):
        pltpu.sync_copy(tabt_hbm.at[pl.ds(b * 8 * VPAD, 8 * VPAD)], tab_v)

        def chunk_pair(r, carry):
            for s in range(NSTG):
                ch = r * NSTG + s
                pltpu.make_async_copy(
                    stg_v.at[s], out_rect(b, ch - NSTG), sems[s]).wait()
                compute_tile(s, ch)
                pltpu.make_async_copy(
                    stg_v.at[s], out_rect(b, ch), sems[s]).start()
            return carry

        for s in range(NSTG):
            compute_tile(s, s)
            pltpu.make_async_copy(
                stg_v.at[s], out_rect(b, s), sems[s]).start()
        lax.fori_loop(1, nch // NSTG, chunk_pair, 0)
        for s in range(NSTG):
            pltpu.make_async_copy(
                stg_v.at[s], out_rect(b, nch - NSTG + s), sems[s]).wait()

    def all_bands(t, carry):
        b = t * NW + wid
        band_loop(b)
        return carry

    full_rounds = NBANDS // NW
    lax.fori_loop(0, full_rounds, all_bands, 0)
    @pl.when(wid < NBANDS - full_rounds * NW)
    def _():
        band_loop(full_rounds * NW + wid)


def _sc_gather_t(idx_f, tabt_flat):
    n = idx_f.shape[0]
    mesh = plsc.VectorSubcoreMesh(
        core_axis_name="c", subcore_axis_name="s",
        num_cores=NC, num_subcores=NS)
    f = pl.kernel(
        _gather_t_body,
        out_type=jax.ShapeDtypeStruct((VOCAB, n), jnp.float32),
        mesh=mesh,
        compiler_params=pltpu.CompilerParams(needs_layout_passes=False),
        scratch_types=[
            pltpu.VMEM((n,), jnp.int32),
            pltpu.VMEM((8 * VPAD,), jnp.float32),
            pltpu.VMEM((NSTG, 8, ICH), jnp.float32),
            [pltpu.SemaphoreType.DMA] * NSTG,
        ],
    )
    return f(idx_f, tabt_flat)


def _loss_body(idx_hbm, tgt_hbm, tab8_hbm, lse_hbm, part_hbm,
               idx_v, tgt_v, fi_v, vals_v, lse_v, acc_v, sem):
    n_per = idx_v.shape[0]
    wid = lax.axis_index("s") * NC + lax.axis_index("c")
    base = wid * n_per

    pltpu.sync_copy(idx_hbm.at[pl.ds(base, n_per)], idx_v)
    pltpu.sync_copy(tgt_hbm.at[pl.ds(base, n_per)], tgt_v)
    pltpu.sync_copy(lse_hbm, lse_v)
    acc_v[...] = jnp.zeros((L,), jnp.float32)

    for j in range(n_per // L):
        i16 = idx_v[pl.ds(j * L, L)]
        t16 = tgt_v[pl.ds(j * L, L)]
        fi = i16 * VOCAB + t16
        fi_v[pl.ds(j * L, L)] = fi >> 3

    pltpu.make_async_copy(tab8_hbm.at[fi_v], vals_v, sem).start()
    pltpu.make_async_copy(tab8_hbm.at[fi_v], vals_v, sem).wait()

    pos0 = jax.lax.broadcasted_iota(jnp.int32, (L,), 0)
    for j in range(n_per // L):
        i16 = idx_v[pl.ds(j * L, L)]
        t16 = tgt_v[pl.ds(j * L, L)]
        fi = i16 * VOCAB + t16
        rem = fi & 7
        v16 = plsc.load_gather(vals_v, [pos0 + j * L, rem])
        lse16 = plsc.load_gather(lse_v, [i16])
        acc_v[...] = acc_v[...] + (lse16 - v16)

    pltpu.sync_copy(acc_v, part_hbm.at[wid])


def _sc_loss(idx_f, tgt_f, tab8, lse_pad):
    n = idx_f.shape[0]
    n_per = n // NW
    mesh = plsc.VectorSubcoreMesh(
        core_axis_name="c", subcore_axis_name="s",
        num_cores=NC, num_subcores=NS)
    f = pl.kernel(
        _loss_body,
        out_type=jax.ShapeDtypeStruct((NW, L), jnp.float32),
        mesh=mesh,
        compiler_params=pltpu.CompilerParams(
            needs_layout_passes=False, use_tc_tiling_on_sc=False),
        scratch_types=[
            pltpu.VMEM((n_per,), jnp.int32),
            pltpu.VMEM((n_per,), jnp.int32),
            pltpu.VMEM((n_per,), jnp.int32),
            pltpu.VMEM((n_per, 8), jnp.float32),
            pltpu.VMEM((VPAD,), jnp.float32),
            pltpu.VMEM((L,), jnp.float32),
            pltpu.SemaphoreType.DMA,
        ],
    )
    return f(idx_f, tgt_f, tab8, lse_pad)


def kernel(idx, targets, table):
    idx_f = idx.reshape(-1)
    tgt_f = targets.reshape(-1)
    lse = _row_lse(table)
    lse_pad = jnp.concatenate(
        [lse[:, 0], jnp.zeros((VPAD - VOCAB,), jnp.float32)])
    tabt_flat = jnp.pad(table.T, ((0, 0), (0, VPAD - VOCAB))).reshape(-1)
    outt = _sc_gather_t(idx_f, tabt_flat)
    tab8 = table.reshape(VOCAB * VOCAB // 8, 8)
    part = _sc_loss(idx_f, tgt_f, tab8, lse_pad)
    loss = jnp.sum(part) / jnp.float32(idx_f.shape[0])
    return (outt.T, loss)


# speedup vs baseline: 2.7782x; 2.7782x over previous
import functools

import jax
import jax.numpy as jnp
from jax import lax
from jax.experimental import pallas as pl
from jax.experimental.pallas import tpu as pltpu
from jax.experimental.pallas import tpu_sc as plsc

NC = 2
NS = 16
NW = NC * NS
L = 16

VOCAB = 1000
VPAD = 1024
NBANDS = VOCAB // 8
ICH = 128
NSTG = 2


def _lse_body(tab_ref, lse_ref):
    x = tab_ref[...]
    m = jnp.max(x, axis=1, keepdims=True)
    s = jnp.sum(jnp.exp(x - m), axis=1, keepdims=True)
    lse_ref[...] = m + jnp.log(s)


def _row_lse(table):
    return pl.pallas_call(
        _lse_body,
        out_shape=jax.ShapeDtypeStruct((VOCAB, 1), jnp.float32),
    )(table)


def _gather_t_body(idx_hbm, tabt_hbm, outt_hbm, idx_v, tab_v, stg_v, sems):
    n = idx_v.shape[0]
    nch = n // ICH
    wid = lax.axis_index("s") * NC + lax.axis_index("c")

    pltpu.sync_copy(idx_hbm, idx_v)

    def out_rect(b, ch):
        return outt_hbm.at[pl.ds(b * 8, 8), pl.ds(ch * ICH, ICH)]

    tab_rows = [tab_v.at[pl.ds(c * VPAD, VPAD)] for c in range(8)]

    def compute_tile(s, ch):
        @plsc.parallel_loop(0, ICH // L, unroll=ICH // L)
        def _(g):
            i16 = idx_v[pl.ds(ch * ICH + g * L, L)]
            for c in range(8):
                v16 = plsc.load_gather(tab_rows[c], [i16])
                stg_v[s, c, pl.ds(g * L, L)] = v16

    def band_loop(b):
        pltpu.sync_copy(tabt_hbm.at[pl.ds(b * 8 * VPAD, 8 * VPAD)], tab_v)

        def chunk_pair(r, carry):
            for s in range(NSTG):
                ch = r * NSTG + s
                pltpu.make_async_copy(
                    stg_v.at[s], out_rect(b, ch - NSTG), sems[s]).wait()
                compute_tile(s, ch)
                pltpu.make_async_copy(
                    stg_v.at[s], out_rect(b, ch), sems[s]).start()
            return carry

        for s in range(NSTG):
            compute_tile(s, s)
            pltpu.make_async_copy(
                stg_v.at[s], out_rect(b, s), sems[s]).start()
        lax.fori_loop(1, nch // NSTG, chunk_pair, 0)
        for s in range(NSTG):
            pltpu.make_async_copy(
                stg_v.at[s], out_rect(b, nch - NSTG + s), sems[s]).wait()

    def all_bands(t, carry):
        b = t * NW + wid
        band_loop(b)
        return carry

    full_rounds = NBANDS // NW
    lax.fori_loop(0, full_rounds, all_bands, 0)
    @pl.when(wid < NBANDS - full_rounds * NW)
    def _():
        band_loop(full_rounds * NW + wid)


def _sc_gather_t(idx_f, tabt_flat):
    n = idx_f.shape[0]
    mesh = plsc.VectorSubcoreMesh(
        core_axis_name="c", subcore_axis_name="s",
        num_cores=NC, num_subcores=NS)
    f = pl.kernel(
        _gather_t_body,
        out_type=jax.ShapeDtypeStruct((VOCAB, n), jnp.float32),
        mesh=mesh,
        compiler_params=pltpu.CompilerParams(needs_layout_passes=False),
        scratch_types=[
            pltpu.VMEM((n,), jnp.int32),
            pltpu.VMEM((8 * VPAD,), jnp.float32),
            pltpu.VMEM((NSTG, 8, ICH), jnp.float32),
            [pltpu.SemaphoreType.DMA] * NSTG,
        ],
    )
    return f(idx_f, tabt_flat)


def _loss_body(idx_hbm, tgt_hbm, tab8_hbm, lse_hbm, part_hbm,
               idx_v, tgt_v, fi_v, vals_v, lse_v, acc_v, sem):
    n_per = idx_v.shape[0]
    wid = lax.axis_index("s") * NC + lax.axis_index("c")
    base = wid * n_per

    pltpu.sync_copy(idx_hbm.at[pl.ds(base, n_per)], idx_v)
    pltpu.sync_copy(tgt_hbm.at[pl.ds(base, n_per)], tgt_v)
    pltpu.sync_copy(lse_hbm, lse_v)
    acc_v[...] = jnp.zeros((L,), jnp.float32)

    for j in range(n_per // L):
        i16 = idx_v[pl.ds(j * L, L)]
        t16 = tgt_v[pl.ds(j * L, L)]
        fi = i16 * VOCAB + t16
        fi_v[pl.ds(j * L, L)] = fi >> 3

    pltpu.make_async_copy(tab8_hbm.at[fi_v], vals_v, sem).start()
    pltpu.make_async_copy(tab8_hbm.at[fi_v], vals_v, sem).wait()

    pos0 = jax.lax.broadcasted_iota(jnp.int32, (L,), 0)
    for j in range(n_per // L):
        i16 = idx_v[pl.ds(j * L, L)]
        t16 = tgt_v[pl.ds(j * L, L)]
        fi = i16 * VOCAB + t16
        rem = fi & 7
        v16 = plsc.load_gather(vals_v, [pos0 + j * L, rem])
        lse16 = plsc.load_gather(lse_v, [i16])
        acc_v[...] = acc_v[...] + (lse16 - v16)

    pltpu.sync_copy(acc_v, part_hbm.at[wid])


def _sc_loss(idx_f, tgt_f, tab8, lse_pad):
    n = idx_f.shape[0]
    n_per = n // NW
    mesh = plsc.VectorSubcoreMesh(
        core_axis_name="c", subcore_axis_name="s",
        num_cores=NC, num_subcores=NS)
    f = pl.kernel(
        _loss_body,
        out_type=jax.ShapeDtypeStruct((NW, L), jnp.float32),
        mesh=mesh,
        compiler_params=pltpu.CompilerParams(
            needs_layout_passes=False, use_tc_tiling_on_sc=False),
        scratch_types=[
            pltpu.VMEM((n_per,), jnp.int32),
            pltpu.VMEM((n_per,), jnp.int32),
            pltpu.VMEM((n_per,), jnp.int32),
            pltpu.VMEM((n_per, 8), jnp.float32),
            pltpu.VMEM((VPAD,), jnp.float32),
            pltpu.VMEM((L,), jnp.float32),
            pltpu.SemaphoreType.DMA,
        ],
    )
    return f(idx_f, tgt_f, tab8, lse_pad)


def kernel(idx, targets, table):
    idx_f = idx.reshape(-1)
    tgt_f = targets.reshape(-1)
    lse = _row_lse(table)
    lse_pad = jnp.concatenate(
        [lse[:, 0], jnp.zeros((VPAD - VOCAB,), jnp.float32)])
    tabt_flat = jnp.pad(table.T, ((0, 0), (0, VPAD - VOCAB))).reshape(-1)
    outt = _sc_gather_t(idx_f, tabt_flat)
    tab8 = table.reshape(VOCAB * VOCAB // 8, 8)
    part = _sc_loss(idx_f, tgt_f, tab8, lse_pad)
    loss = jnp.sum(part) / jnp.float32(idx_f.shape[0])
    return (outt.T, loss)


# speedup vs baseline: 3.5150x; 1.2652x over previous
import functools

import jax
import jax.numpy as jnp
from jax import lax
from jax.experimental import pallas as pl
from jax.experimental.pallas import tpu as pltpu
from jax.experimental.pallas import tpu_sc as plsc

NC = 2
NS = 16
NW = NC * NS
L = 16

VOCAB = 1000
VPAD = 1024
NBANDS = VOCAB // 8
ICH = 512
NSTG = 4


def _lse_body(tab_ref, lse_ref):
    x = tab_ref[...]
    m = jnp.max(x, axis=1, keepdims=True)
    s = jnp.sum(jnp.exp(x - m), axis=1, keepdims=True)
    lse_ref[...] = m + jnp.log(s)


def _row_lse(table):
    return pl.pallas_call(
        _lse_body,
        out_shape=jax.ShapeDtypeStruct((VOCAB, 1), jnp.float32),
    )(table)


def _gather_t_body(idx_hbm, tabt_hbm, outt_hbm, idx_v, tab_v, stg_v, sems):
    n = idx_v.shape[0]
    nch = n // ICH
    wid = lax.axis_index("s") * NC + lax.axis_index("c")

    pltpu.sync_copy(idx_hbm, idx_v)

    def out_rect(b, ch):
        return outt_hbm.at[pl.ds(b * 8, 8), pl.ds(ch * ICH, ICH)]

    tab_rows = [tab_v.at[pl.ds(c * VPAD, VPAD)] for c in range(8)]

    def compute_tile(s, ch):
        @plsc.parallel_loop(0, ICH // L, unroll=8)
        def _(g):
            i16 = idx_v[pl.ds(ch * ICH + g * L, L)]
            for c in range(8):
                v16 = plsc.load_gather(tab_rows[c], [i16])
                stg_v[s, c, pl.ds(g * L, L)] = v16

    def band_loop(b):
        pltpu.sync_copy(tabt_hbm.at[pl.ds(b * 8 * VPAD, 8 * VPAD)], tab_v)

        def chunk_pair(r, carry):
            for s in range(NSTG):
                ch = r * NSTG + s
                pltpu.make_async_copy(
                    stg_v.at[s], out_rect(b, ch - NSTG), sems[s]).wait()
                compute_tile(s, ch)
                pltpu.make_async_copy(
                    stg_v.at[s], out_rect(b, ch), sems[s]).start()
            return carry

        for s in range(NSTG):
            compute_tile(s, s)
            pltpu.make_async_copy(
                stg_v.at[s], out_rect(b, s), sems[s]).start()
        lax.fori_loop(1, nch // NSTG, chunk_pair, 0)
        for s in range(NSTG):
            pltpu.make_async_copy(
                stg_v.at[s], out_rect(b, nch - NSTG + s), sems[s]).wait()

    def all_bands(t, carry):
        b = t * NW + wid
        band_loop(b)
        return carry

    full_rounds = NBANDS // NW
    lax.fori_loop(0, full_rounds, all_bands, 0)
    @pl.when(wid < NBANDS - full_rounds * NW)
    def _():
        band_loop(full_rounds * NW + wid)


def _sc_gather_t(idx_f, tabt_flat):
    n = idx_f.shape[0]
    mesh = plsc.VectorSubcoreMesh(
        core_axis_name="c", subcore_axis_name="s",
        num_cores=NC, num_subcores=NS)
    f = pl.kernel(
        _gather_t_body,
        out_type=jax.ShapeDtypeStruct((VOCAB, n), jnp.float32),
        mesh=mesh,
        compiler_params=pltpu.CompilerParams(needs_layout_passes=False),
        scratch_types=[
            pltpu.VMEM((n,), jnp.int32),
            pltpu.VMEM((8 * VPAD,), jnp.float32),
            pltpu.VMEM((NSTG, 8, ICH), jnp.float32),
            [pltpu.SemaphoreType.DMA] * NSTG,
        ],
    )
    return f(idx_f, tabt_flat)


def _loss_body(idx_hbm, tgt_hbm, tab8_hbm, lse_hbm, part_hbm,
               idx_v, tgt_v, fi_v, vals_v, lse_v, acc_v, sem):
    n_per = idx_v.shape[0]
    wid = lax.axis_index("s") * NC + lax.axis_index("c")
    base = wid * n_per

    pltpu.sync_copy(idx_hbm.at[pl.ds(base, n_per)], idx_v)
    pltpu.sync_copy(tgt_hbm.at[pl.ds(base, n_per)], tgt_v)
    pltpu.sync_copy(lse_hbm, lse_v)
    acc_v[...] = jnp.zeros((L,), jnp.float32)

    for j in range(n_per // L):
        i16 = idx_v[pl.ds(j * L, L)]
        t16 = tgt_v[pl.ds(j * L, L)]
        fi = i16 * VOCAB + t16
        fi_v[pl.ds(j * L, L)] = fi >> 3

    pltpu.make_async_copy(tab8_hbm.at[fi_v], vals_v, sem).start()
    pltpu.make_async_copy(tab8_hbm.at[fi_v], vals_v, sem).wait()

    pos0 = jax.lax.broadcasted_iota(jnp.int32, (L,), 0)
    for j in range(n_per // L):
        i16 = idx_v[pl.ds(j * L, L)]
        t16 = tgt_v[pl.ds(j * L, L)]
        fi = i16 * VOCAB + t16
        rem = fi & 7
        v16 = plsc.load_gather(vals_v, [pos0 + j * L, rem])
        lse16 = plsc.load_gather(lse_v, [i16])
        acc_v[...] = acc_v[...] + (lse16 - v16)

    pltpu.sync_copy(acc_v, part_hbm.at[wid])


def _sc_loss(idx_f, tgt_f, tab8, lse_pad):
    n = idx_f.shape[0]
    n_per = n // NW
    mesh = plsc.VectorSubcoreMesh(
        core_axis_name="c", subcore_axis_name="s",
        num_cores=NC, num_subcores=NS)
    f = pl.kernel(
        _loss_body,
        out_type=jax.ShapeDtypeStruct((NW, L), jnp.float32),
        mesh=mesh,
        compiler_params=pltpu.CompilerParams(
            needs_layout_passes=False, use_tc_tiling_on_sc=False),
        scratch_types=[
            pltpu.VMEM((n_per,), jnp.int32),
            pltpu.VMEM((n_per,), jnp.int32),
            pltpu.VMEM((n_per,), jnp.int32),
            pltpu.VMEM((n_per, 8), jnp.float32),
            pltpu.VMEM((VPAD,), jnp.float32),
            pltpu.VMEM((L,), jnp.float32),
            pltpu.SemaphoreType.DMA,
        ],
    )
    return f(idx_f, tgt_f, tab8, lse_pad)


def kernel(idx, targets, table):
    idx_f = idx.reshape(-1)
    tgt_f = targets.reshape(-1)
    lse = _row_lse(table)
    lse_pad = jnp.concatenate(
        [lse[:, 0], jnp.zeros((VPAD - VOCAB,), jnp.float32)])
    tabt_flat = jnp.pad(table.T, ((0, 0), (0, VPAD - VOCAB))).reshape(-1)
    outt = _sc_gather_t(idx_f, tabt_flat)
    tab8 = table.reshape(VOCAB * VOCAB // 8, 8)
    part = _sc_loss(idx_f, tgt_f, tab8, lse_pad)
    loss = jnp.sum(part) / jnp.float32(idx_f.shape[0])
    return (outt.T, loss)
